# baseline (speedup 1.0000x reference)
import dataclasses
import functools

import jax
import jax.numpy as jnp
from jax.experimental import pallas as pl
from jax.experimental.pallas import tpu as pltpu
from jax.experimental.pallas import tpu_sc as plsc

HIDDEN = 128
EPS = 1e-12
GATHER_WINDOW = 128
BB = 8


KWIN = 2


def _sc_gather(table, flat_ids):
    n = flat_ids.shape[0]
    idx = flat_ids.reshape(1, n // GATHER_WINDOW, GATHER_WINDOW)
    mesh = plsc.VectorSubcoreMesh(core_axis_name="c", subcore_axis_name="s")

    @functools.partial(
        pl.kernel,
        out_type=jax.ShapeDtypeStruct((n, HIDDEN), table.dtype),
        mesh=mesh,
        scratch_types=[pltpu.SemaphoreType.DMA] * KWIN,
    )
    def gather_kernel(table_hbm, idx_hbm, out_hbm, *sems):
        def body(idx_vmem, out_vmem):
            copies = []
            for j in range(KWIN):
                copies.append(pltpu.async_copy(
                    table_hbm.at[idx_vmem.at[0, j]],
                    out_vmem.at[pl.ds(j * GATHER_WINDOW, GATHER_WINDOW)],
                    sems[j]))
            for c in copies:
                c.wait()

        pltpu.emit_pipeline(
            body,
            grid=(n // (KWIN * GATHER_WINDOW),),
            in_specs=[pl.BlockSpec((1, KWIN, GATHER_WINDOW),
                                   index_map=lambda i: (0, i, 0))],
            out_specs=[pl.BlockSpec((KWIN * GATHER_WINDOW, HIDDEN),
                                    index_map=lambda i: (i, 0))],
            core_axis_name=("c", "s"),
            dimension_semantics=(pltpu.PARALLEL,),
        )(idx_hbm, out_hbm)

    return gather_kernel(table, idx)


W = 128
NWORKERS = 32
RSQRT_MAGIC = 0x5F3759DF


def _sc_fused(table, idxpack, comb_flat, lnwb):
    n = idxpack.shape[1]
    tok_per_w = n // NWORKERS
    nwin = tok_per_w // W
    mesh = plsc.VectorSubcoreMesh(core_axis_name="c", subcore_axis_name="s")

    cp = pltpu.CompilerParams()
    if "needs_layout_passes" in pltpu.CompilerParams.__dataclass_fields__:
        cp = dataclasses.replace(cp, needs_layout_passes=False)

    @functools.partial(
        pl.kernel,
        out_type=jax.ShapeDtypeStruct((n, HIDDEN), jnp.float32),
        mesh=mesh,
        compiler_params=cp,
        scratch_types=[
            pltpu.VMEM((400 * HIDDEN,), jnp.float32),
            pltpu.VMEM((256,), jnp.float32),
            pltpu.VMEM((2, W, HIDDEN), jnp.float32),
            pltpu.VMEM((2, W, HIDDEN), jnp.float32),
            pltpu.VMEM((4, 2, W), jnp.int32),
        ] + [pltpu.SemaphoreType.DMA] * 8,
    )
    def fused_kernel(table_hbm, idx_hbm, comb_hbm, lnwb_hbm, out_hbm,
                     comb_v, lnwb_v, work, obuf, widx,
                     g0, g1, o0, o1, i0, i1, i2, i3):
        gsem = (g0, g1)
        osem = (o0, o1)
        isem = (i0, i1, i2, i3)
        c = jax.lax.axis_index("c")
        s = jax.lax.axis_index("s")
        wid = s * 2 + c
        tbase = wid * tok_per_w

        pltpu.sync_copy(comb_hbm, comb_v)
        pltpu.sync_copy(lnwb_hbm, lnwb_v)

        iota = jax.lax.iota(jnp.int32, 16)
        offs = [iota + 16 * j for j in range(8)]
        w_regs = [lnwb_v[pl.ds(16 * j, 16)] for j in range(8)]
        b_regs = [lnwb_v[pl.ds(128 + 16 * j, 16)] for j in range(8)]

        def idx_start(g, b4):
            return pltpu.async_copy(
                idx_hbm.at[:, pl.ds(tbase + g * W, W)], widx.at[b4], isem[b4])

        def i_wait(b4):
            pltpu.make_async_copy(
                idx_hbm.at[:, pl.ds(0, W)], widx.at[b4], isem[b4]).wait()

        def gather_start(b4, b2):
            return pltpu.async_copy(
                table_hbm.at[widx.at[b4, 0]], work.at[b2], gsem[b2])

        def g_wait(b4, b2):
            pltpu.make_async_copy(
                table_hbm.at[widx.at[b4, 0]], work.at[b2], gsem[b2]).wait()

        def out_start(g, b2):
            return pltpu.async_copy(
                obuf.at[b2], out_hbm.at[pl.ds(tbase + g * W, W)], osem[b2])

        def o_wait(b2):
            pltpu.make_async_copy(
                obuf.at[b2], out_hbm.at[pl.ds(0, W)], osem[b2]).wait()

        def compute(b4, b2):
            @pl.loop(0, W)
            def _(t):
                cvec = plsc.load_gather(
                    widx, [jnp.full((16,), b4, jnp.int32),
                           jnp.full((16,), 1, jnp.int32),
                           jnp.full((16,), t, jnp.int32)])
                cbase = cvec * HIDDEN
                v = []
                for j in range(8):
                    xj = work[b2, t, pl.ds(16 * j, 16)]
                    cj = plsc.load_gather(comb_v, [cbase + offs[j]])
                    v.append(xj + cj)
                s0 = (v[0] + v[1]) + (v[2] + v[3])
                s1 = (v[4] + v[5]) + (v[6] + v[7])
                svec = s0 + s1
                q = [vj * vj for vj in v]
                q0 = (q[0] + q[1]) + (q[2] + q[3])
                q1 = (q[4] + q[5]) + (q[6] + q[7])
                qvec = q0 + q1
                tot = jnp.sum(svec)
                tot2 = jnp.sum(qvec)
                mean_v = jnp.full((16,), tot, jnp.float32) * (1.0 / HIDDEN)
                var_v = (jnp.full((16,), tot2, jnp.float32) * (1.0 / HIDDEN)
                         - mean_v * mean_v) + EPS
                yi = RSQRT_MAGIC - (plsc.bitcast(var_v, jnp.int32) >> 1)
                y = plsc.bitcast(yi, jnp.float32)
                hx = var_v * 0.5
                for _it in range(3):
                    y = y * (1.5 - hx * y * y)
                for j in range(8):
                    obuf[b2, t, pl.ds(16 * j, 16)] = (
                        (v[j] - mean_v) * y * w_regs[j] + b_regs[j])

        for g in range(4):
            idx_start(g, g)
        for g in range(2):
            i_wait(g)
            gather_start(g, g)
        for gg in range(4):
            b4, b2 = gg & 3, gg & 1
            g_wait(b4, b2)
            if gg >= 2:
                o_wait(b2)
            compute(b4, b2)
            out_start(gg, b2)
            idx_start(gg + 4, b4)
            i_wait((gg + 2) & 3)
            gather_start((gg + 2) & 3, b2)

        @pl.loop(1, nwin // 4 - 1)
        def _(wq):
            for b in range(4):
                gg = wq * 4 + b
                b4, b2 = b & 3, b & 1
                g_wait(b4, b2)
                o_wait(b2)
                compute(b4, b2)
                out_start(gg, b2)
                idx_start(gg + 4, b4)
                i_wait((b + 2) & 3)
                gather_start((b + 2) & 3, b2)

        for b in range(4):
            gg = nwin - 4 + b
            b4, b2 = gg & 3, gg & 1
            g_wait(b4, b2)
            o_wait(b2)
            compute(b4, b2)
            out_start(gg, b2)
            if b < 2:
                i_wait((gg + 2) & 3)
                gather_start((gg + 2) & 3, b2)
        o_wait(0)
        o_wait(1)

    return fused_kernel(table, idxpack, comb_flat, lnwb)


def _addln_body(g_ref, tt_ref, base_ref, diff_ref, w_ref, b_ref, o_ref):
    x = g_ref[...]
    x = x + base_ref[...] + tt_ref[...] * diff_ref[...]
    mean = jnp.mean(x, axis=-1, keepdims=True)
    xc = x - mean
    var = jnp.mean(xc * xc, axis=-1, keepdims=True)
    inv = jax.lax.rsqrt(var + EPS)
    o_ref[...] = xc * inv * w_ref[...] + b_ref[...]


def _tc_addln(gathered, tt_f, base, diff, ln_w, ln_b, interpret=False):
    b, s, h = gathered.shape
    return pl.pallas_call(
        _addln_body,
        grid=(b // BB,),
        in_specs=[
            pl.BlockSpec((BB, s, h), lambda i: (i, 0, 0)),
            pl.BlockSpec((BB, s, 1), lambda i: (i, 0, 0)),
            pl.BlockSpec((1, s, h), lambda i: (0, 0, 0)),
            pl.BlockSpec((1, 1, h), lambda i: (0, 0, 0)),
            pl.BlockSpec((1, 1, h), lambda i: (0, 0, 0)),
            pl.BlockSpec((1, 1, h), lambda i: (0, 0, 0)),
        ],
        out_specs=pl.BlockSpec((BB, s, h), lambda i: (i, 0, 0)),
        out_shape=jax.ShapeDtypeStruct((b, s, h), jnp.float32),
        interpret=interpret,
    )(gathered, tt_f, base, diff, ln_w, ln_b)


def kernel(input_ids, token_type_ids, W_word, W_pos, W_type, ln_w, ln_b):
    b, s = input_ids.shape
    flat_ids = input_ids.reshape(-1).astype(jnp.int32)
    pos = jnp.arange(s, dtype=jnp.int32)
    cidx = (2 * pos[None, :] + token_type_ids.astype(jnp.int32)).reshape(-1)
    idxpack = jnp.stack([flat_ids, cidx])
    comb_flat = (W_pos[:s, None, :] + W_type[None, :, :]).reshape(-1)
    lnwb = jnp.concatenate([ln_w, ln_b])
    return _sc_fused(W_word, idxpack, comb_flat, lnwb).reshape(b, s, HIDDEN)


# speedup vs baseline: 2.2606x; 2.2606x over previous
import dataclasses
import functools

import jax
import jax.numpy as jnp
from jax.experimental import pallas as pl
from jax.experimental.pallas import tpu as pltpu
from jax.experimental.pallas import tpu_sc as plsc

HIDDEN = 128
EPS = 1e-12
GATHER_WINDOW = 128
BB = 8


KWIN = 2


def _sc_gather(table, flat_ids):
    n = flat_ids.shape[0]
    idx = flat_ids.reshape(1, n // GATHER_WINDOW, GATHER_WINDOW)
    mesh = plsc.VectorSubcoreMesh(core_axis_name="c", subcore_axis_name="s")

    @functools.partial(
        pl.kernel,
        out_type=jax.ShapeDtypeStruct((n, HIDDEN), table.dtype),
        mesh=mesh,
        scratch_types=[pltpu.SemaphoreType.DMA] * KWIN,
    )
    def gather_kernel(table_hbm, idx_hbm, out_hbm, *sems):
        def body(idx_vmem, out_vmem):
            copies = []
            for j in range(KWIN):
                copies.append(pltpu.async_copy(
                    table_hbm.at[idx_vmem.at[0, j]],
                    out_vmem.at[pl.ds(j * GATHER_WINDOW, GATHER_WINDOW)],
                    sems[j]))
            for c in copies:
                c.wait()

        pltpu.emit_pipeline(
            body,
            grid=(n // (KWIN * GATHER_WINDOW),),
            in_specs=[pl.BlockSpec((1, KWIN, GATHER_WINDOW),
                                   index_map=lambda i: (0, i, 0))],
            out_specs=[pl.BlockSpec((KWIN * GATHER_WINDOW, HIDDEN),
                                    index_map=lambda i: (i, 0))],
            core_axis_name=("c", "s"),
            dimension_semantics=(pltpu.PARALLEL,),
        )(idx_hbm, out_hbm)

    return gather_kernel(table, idx)


W = 128
NWORKERS = 32
RSQRT_MAGIC = 0x5F3759DF


def _sc_fused(table, idxpack, comb_flat, lnwb):
    n = idxpack.shape[1]
    tok_per_w = n // NWORKERS
    nwin = tok_per_w // W
    mesh = plsc.VectorSubcoreMesh(core_axis_name="c", subcore_axis_name="s")

    cp = pltpu.CompilerParams()
    if "needs_layout_passes" in pltpu.CompilerParams.__dataclass_fields__:
        cp = dataclasses.replace(cp, needs_layout_passes=False)

    @functools.partial(
        pl.kernel,
        out_type=jax.ShapeDtypeStruct((n, HIDDEN), jnp.float32),
        mesh=mesh,
        compiler_params=cp,
        scratch_types=[
            pltpu.VMEM((400 * HIDDEN,), jnp.float32),
            pltpu.VMEM((256,), jnp.float32),
            pltpu.VMEM((2, W, HIDDEN), jnp.float32),
            pltpu.VMEM((2, W, HIDDEN), jnp.float32),
            pltpu.VMEM((4, 2, W), jnp.int32),
        ] + [pltpu.SemaphoreType.DMA] * 8,
    )
    def fused_kernel(table_hbm, idx_hbm, comb_hbm, lnwb_hbm, out_hbm,
                     comb_v, lnwb_v, work, obuf, widx,
                     g0, g1, o0, o1, i0, i1, i2, i3):
        gsem = (g0, g1)
        osem = (o0, o1)
        isem = (i0, i1, i2, i3)
        c = jax.lax.axis_index("c")
        s = jax.lax.axis_index("s")
        wid = s * 2 + c
        tbase = wid * tok_per_w

        pltpu.sync_copy(comb_hbm, comb_v)
        pltpu.sync_copy(lnwb_hbm, lnwb_v)

        iota = jax.lax.iota(jnp.int32, 16)
        offs = [iota + 16 * j for j in range(8)]
        w_regs = [lnwb_v[pl.ds(16 * j, 16)] for j in range(8)]
        b_regs = [lnwb_v[pl.ds(128 + 16 * j, 16)] for j in range(8)]

        def idx_start(g, b4):
            return pltpu.async_copy(
                idx_hbm.at[:, pl.ds(tbase + g * W, W)], widx.at[b4], isem[b4])

        def i_wait(b4):
            pltpu.make_async_copy(
                idx_hbm.at[:, pl.ds(0, W)], widx.at[b4], isem[b4]).wait()

        def gather_start(b4, b2):
            return pltpu.async_copy(
                table_hbm.at[widx.at[b4, 0]], work.at[b2], gsem[b2])

        def g_wait(b4, b2):
            pltpu.make_async_copy(
                table_hbm.at[widx.at[b4, 0]], work.at[b2], gsem[b2]).wait()

        def out_start(g, b2):
            return pltpu.async_copy(
                obuf.at[b2], out_hbm.at[pl.ds(tbase + g * W, W)], osem[b2])

        def o_wait(b2):
            pltpu.make_async_copy(
                obuf.at[b2], out_hbm.at[pl.ds(0, W)], osem[b2]).wait()

        def compute(b4, b2):
            @plsc.parallel_loop(0, W, unroll=2)
            def _(t):
                cvec = plsc.load_gather(
                    widx, [jnp.full((16,), b4, jnp.int32),
                           jnp.full((16,), 1, jnp.int32),
                           jnp.full((16,), t, jnp.int32)])
                cbase = cvec * HIDDEN
                v = []
                for j in range(8):
                    xj = work[b2, t, pl.ds(16 * j, 16)]
                    cj = plsc.load_gather(comb_v, [cbase + offs[j]])
                    v.append(xj + cj)
                s0 = (v[0] + v[1]) + (v[2] + v[3])
                s1 = (v[4] + v[5]) + (v[6] + v[7])
                svec = s0 + s1
                q = [vj * vj for vj in v]
                q0 = (q[0] + q[1]) + (q[2] + q[3])
                q1 = (q[4] + q[5]) + (q[6] + q[7])
                qvec = q0 + q1
                tot = jnp.sum(svec)
                tot2 = jnp.sum(qvec)
                mean_v = jnp.full((16,), tot, jnp.float32) * (1.0 / HIDDEN)
                var_v = (jnp.full((16,), tot2, jnp.float32) * (1.0 / HIDDEN)
                         - mean_v * mean_v) + EPS
                yi = RSQRT_MAGIC - (plsc.bitcast(var_v, jnp.int32) >> 1)
                y = plsc.bitcast(yi, jnp.float32)
                hx = var_v * 0.5
                for _it in range(2):
                    y = y * (1.5 - hx * y * y)
                for j in range(8):
                    obuf[b2, t, pl.ds(16 * j, 16)] = (
                        (v[j] - mean_v) * y * w_regs[j] + b_regs[j])

        for g in range(4):
            idx_start(g, g)
        for g in range(2):
            i_wait(g)
            gather_start(g, g)
        for gg in range(4):
            b4, b2 = gg & 3, gg & 1
            g_wait(b4, b2)
            if gg >= 2:
                o_wait(b2)
            compute(b4, b2)
            out_start(gg, b2)
            idx_start(gg + 4, b4)
            i_wait((gg + 2) & 3)
            gather_start((gg + 2) & 3, b2)

        @pl.loop(1, nwin // 4 - 1)
        def _(wq):
            for b in range(4):
                gg = wq * 4 + b
                b4, b2 = b & 3, b & 1
                g_wait(b4, b2)
                o_wait(b2)
                compute(b4, b2)
                out_start(gg, b2)
                idx_start(gg + 4, b4)
                i_wait((b + 2) & 3)
                gather_start((b + 2) & 3, b2)

        for b in range(4):
            gg = nwin - 4 + b
            b4, b2 = gg & 3, gg & 1
            g_wait(b4, b2)
            o_wait(b2)
            compute(b4, b2)
            out_start(gg, b2)
            if b < 2:
                i_wait((gg + 2) & 3)
                gather_start((gg + 2) & 3, b2)
        o_wait(0)
        o_wait(1)

    return fused_kernel(table, idxpack, comb_flat, lnwb)


def _addln_body(g_ref, tt_ref, base_ref, diff_ref, w_ref, b_ref, o_ref):
    x = g_ref[...]
    x = x + base_ref[...] + tt_ref[...] * diff_ref[...]
    mean = jnp.mean(x, axis=-1, keepdims=True)
    xc = x - mean
    var = jnp.mean(xc * xc, axis=-1, keepdims=True)
    inv = jax.lax.rsqrt(var + EPS)
    o_ref[...] = xc * inv * w_ref[...] + b_ref[...]


def _tc_addln(gathered, tt_f, base, diff, ln_w, ln_b, interpret=False):
    b, s, h = gathered.shape
    return pl.pallas_call(
        _addln_body,
        grid=(b // BB,),
        in_specs=[
            pl.BlockSpec((BB, s, h), lambda i: (i, 0, 0)),
            pl.BlockSpec((BB, s, 1), lambda i: (i, 0, 0)),
            pl.BlockSpec((1, s, h), lambda i: (0, 0, 0)),
            pl.BlockSpec((1, 1, h), lambda i: (0, 0, 0)),
            pl.BlockSpec((1, 1, h), lambda i: (0, 0, 0)),
            pl.BlockSpec((1, 1, h), lambda i: (0, 0, 0)),
        ],
        out_specs=pl.BlockSpec((BB, s, h), lambda i: (i, 0, 0)),
        out_shape=jax.ShapeDtypeStruct((b, s, h), jnp.float32),
        interpret=interpret,
    )(gathered, tt_f, base, diff, ln_w, ln_b)


def kernel(input_ids, token_type_ids, W_word, W_pos, W_type, ln_w, ln_b):
    b, s = input_ids.shape
    flat_ids = input_ids.reshape(-1).astype(jnp.int32)
    pos = jnp.arange(s, dtype=jnp.int32)
    cidx = (2 * pos[None, :] + token_type_ids.astype(jnp.int32)).reshape(-1)
    idxpack = jnp.stack([flat_ids, cidx])
    comb_flat = (W_pos[:s, None, :] + W_type[None, :, :]).reshape(-1)
    lnwb = jnp.concatenate([ln_w, ln_b])
    return _sc_fused(W_word, idxpack, comb_flat, lnwb).reshape(b, s, HIDDEN)
